# baseline (speedup 1.0000x reference)
import functools

import jax
import jax.numpy as jnp
from jax import lax
from jax.experimental import pallas as pl
from jax.experimental.pallas import tpu as pltpu
from jax.experimental.pallas import tpu_sc as plsc

B, K, C, H, W = 8, 8, 3, 384, 384
HW = H * W
NWORKERS = 32
ROWS_PER_WORKER = (B * H) // NWORKERS
NH = 4
NCHUNKS = ROWS_PER_WORKER // NH
NPIX = NH * W
LANES = 16
STEPS = NPIX // LANES
WORKERS_PER_B = H // ROWS_PER_WORKER

f32 = jnp.float32


def _sc_body(cand_hbm, scores_hbm, mask_hbm, partial_hbm,
             final_hbm, weights_hbm,
             sv, cv, mv, pv, wv, fv):
    cid = lax.axis_index("c")
    sid = lax.axis_index("s")
    wid = cid * 16 + sid
    b = wid // WORKERS_PER_B
    h0 = (wid % WORKERS_PER_B) * ROWS_PER_WORKER

    def chunk_body(t, carry):
        base = (h0 + t * NH) * W
        pltpu.sync_copy(scores_hbm.at[b, :, pl.ds(base, NPIX)], sv)
        pltpu.sync_copy(cand_hbm.at[b, :, pl.ds(base, NPIX)], cv)
        pltpu.sync_copy(mask_hbm.at[b, pl.ds(base, NPIX)], mv)
        pltpu.sync_copy(partial_hbm.at[b, :, pl.ds(base, NPIX)], pv)

        ones = jnp.full((LANES,), 1.0, f32)
        zeros = jnp.full((LANES,), 0.0, f32)

        def step(i, carry2):
            off = i * LANES
            sl = pl.ds(off, LANES)
            m0 = sv[0, sl]
            idx = jnp.zeros((LANES,), jnp.int32)
            vals = [cv[c, sl] for c in range(C)]
            for k in range(1, K):
                vk = sv[k, sl]
                gt = vk > m0
                m0 = jnp.where(gt, vk, m0)
                idx = jnp.where(gt, jnp.full((LANES,), k, jnp.int32), idx)
                for c in range(C):
                    vals[c] = jnp.where(gt, cv[k * C + c, sl], vals[c])
            for k in range(K):
                kvec = jnp.full((LANES,), k, jnp.int32)
                wv[k, sl] = jnp.where(idx == kvec, ones, zeros)
            mvec = mv[sl]
            fmv = 1.0 - mvec
            for c in range(C):
                fv[c, sl] = pv[c, sl] * mvec + vals[c] * fmv
            return carry2

        lax.fori_loop(0, STEPS, step, 0)

        pltpu.sync_copy(wv, weights_hbm.at[b, :, pl.ds(base, NPIX)])
        pltpu.sync_copy(fv, final_hbm.at[b, :, pl.ds(base, NPIX)])
        return carry

    lax.fori_loop(0, NCHUNKS, chunk_body, 0)


@jax.jit
def _run(candidate_images, selection_scores, mask, partial_image):
    cand = candidate_images.reshape(B, K * C, HW)
    scores = selection_scores.reshape(B, K, HW)
    m = mask.reshape(B, HW)
    partial = partial_image.reshape(B, C, HW)

    mesh = plsc.VectorSubcoreMesh(core_axis_name="c", subcore_axis_name="s")
    sc = functools.partial(
        pl.kernel,
        mesh=mesh,
        out_type=(
            jax.ShapeDtypeStruct((B, C, HW), f32),
            jax.ShapeDtypeStruct((B, K, HW), f32),
        ),
        scratch_types=[
            pltpu.VMEM((K, NPIX), f32),
            pltpu.VMEM((K * C, NPIX), f32),
            pltpu.VMEM((NPIX,), f32),
            pltpu.VMEM((C, NPIX), f32),
            pltpu.VMEM((K, NPIX), f32),
            pltpu.VMEM((C, NPIX), f32),
        ],
    )(_sc_body)
    final, weights = sc(cand, scores, m, partial)
    return (final.reshape(B, C, H, W), weights.reshape(B, K, H, W))


def kernel(candidate_images, selection_scores, mask, partial_image):
    return _run(candidate_images, selection_scores, mask, partial_image)


# speedup vs baseline: 1.2863x; 1.2863x over previous
import functools

import jax
import jax.numpy as jnp
from jax import lax
from jax.experimental import pallas as pl
from jax.experimental.pallas import tpu as pltpu
from jax.experimental.pallas import tpu_sc as plsc

B, K, C, H, W = 8, 8, 3, 384, 384
HW = H * W
NWORKERS = 32
ROWS_PER_WORKER = (B * H) // NWORKERS
NH = 3
NCHUNKS = ROWS_PER_WORKER // NH
NPIX = NH * W
LANES = 16
STEPS = NPIX // LANES
WORKERS_PER_B = H // ROWS_PER_WORKER

f32 = jnp.float32


def _sc_body(cand_hbm, scores_hbm, mask_hbm, partial_hbm,
             final_hbm, weights_hbm,
             sv0, sv1, cv0, cv1, mv0, mv1, pv0, pv1,
             wv0, wv1, fv0, fv1,
             isem0, isem1, osem0, osem1):
    cid = lax.axis_index("c")
    sid = lax.axis_index("s")
    wid = cid * 16 + sid
    b = wid // WORKERS_PER_B
    base0 = (wid % WORKERS_PER_B) * ROWS_PER_WORKER * W

    svs, cvs, mvs, pvs = (sv0, sv1), (cv0, cv1), (mv0, mv1), (pv0, pv1)
    wvs, fvs = (wv0, wv1), (fv0, fv1)
    isems, osems = (isem0, isem1), (osem0, osem1)

    ones = jnp.full((LANES,), 1.0, f32)
    zeros = jnp.full((LANES,), 0.0, f32)

    def issue_in(t):
        slot = t % 2
        base = base0 + t * NPIX
        return [
            pltpu.async_copy(scores_hbm.at[b, :, pl.ds(base, NPIX)],
                             svs[slot], isems[slot]),
            pltpu.async_copy(cand_hbm.at[b, :, pl.ds(base, NPIX)],
                             cvs[slot], isems[slot]),
            pltpu.async_copy(mask_hbm.at[b, pl.ds(base, NPIX)],
                             mvs[slot], isems[slot]),
            pltpu.async_copy(partial_hbm.at[b, :, pl.ds(base, NPIX)],
                             pvs[slot], isems[slot]),
        ]

    def issue_out(t):
        slot = t % 2
        base = base0 + t * NPIX
        return [
            pltpu.async_copy(wvs[slot], weights_hbm.at[b, :, pl.ds(base, NPIX)],
                             osems[slot]),
            pltpu.async_copy(fvs[slot], final_hbm.at[b, :, pl.ds(base, NPIX)],
                             osems[slot]),
        ]

    def compute(slot):
        sv, cv, mv, pv = svs[slot], cvs[slot], mvs[slot], pvs[slot]
        wv, fv = wvs[slot], fvs[slot]

        def step(i, carry2):
            off = i * LANES
            sl = pl.ds(off, LANES)
            m0 = sv[0, sl]
            idx = jnp.zeros((LANES,), jnp.int32)
            vals = [cv[c, sl] for c in range(C)]
            for k in range(1, K):
                vk = sv[k, sl]
                gt = vk > m0
                m0 = jnp.where(gt, vk, m0)
                idx = jnp.where(gt, jnp.full((LANES,), k, jnp.int32), idx)
                for c in range(C):
                    vals[c] = jnp.where(gt, cv[k * C + c, sl], vals[c])
            for k in range(K):
                kvec = jnp.full((LANES,), k, jnp.int32)
                wv[k, sl] = jnp.where(idx == kvec, ones, zeros)
            mvec = mv[sl]
            fmv = 1.0 - mvec
            for c in range(C):
                fv[c, sl] = pv[c, sl] * mvec + vals[c] * fmv
            return carry2

        lax.fori_loop(0, STEPS, step, 0)

    out_descs = [None, None]
    in_descs = issue_in(0)
    for t in range(NCHUNKS):
        slot = t % 2
        next_descs = issue_in(t + 1) if t + 1 < NCHUNKS else None
        for d in in_descs:
            d.wait()
        if out_descs[slot] is not None:
            for d in out_descs[slot]:
                d.wait()
        compute(slot)
        out_descs[slot] = issue_out(t)
        in_descs = next_descs
    for slot in range(2):
        if out_descs[slot] is not None:
            for d in out_descs[slot]:
                d.wait()


@jax.jit
def _run(candidate_images, selection_scores, mask, partial_image):
    cand = candidate_images.reshape(B, K * C, HW)
    scores = selection_scores.reshape(B, K, HW)
    m = mask.reshape(B, HW)
    partial = partial_image.reshape(B, C, HW)

    mesh = plsc.VectorSubcoreMesh(core_axis_name="c", subcore_axis_name="s")
    sc = functools.partial(
        pl.kernel,
        mesh=mesh,
        out_type=(
            jax.ShapeDtypeStruct((B, C, HW), f32),
            jax.ShapeDtypeStruct((B, K, HW), f32),
        ),
        scratch_types=[
            pltpu.VMEM((K, NPIX), f32), pltpu.VMEM((K, NPIX), f32),
            pltpu.VMEM((K * C, NPIX), f32), pltpu.VMEM((K * C, NPIX), f32),
            pltpu.VMEM((NPIX,), f32), pltpu.VMEM((NPIX,), f32),
            pltpu.VMEM((C, NPIX), f32), pltpu.VMEM((C, NPIX), f32),
            pltpu.VMEM((K, NPIX), f32), pltpu.VMEM((K, NPIX), f32),
            pltpu.VMEM((C, NPIX), f32), pltpu.VMEM((C, NPIX), f32),
            pltpu.SemaphoreType.DMA, pltpu.SemaphoreType.DMA,
            pltpu.SemaphoreType.DMA, pltpu.SemaphoreType.DMA,
        ],
    )(_sc_body)
    final, weights = sc(cand, scores, m, partial)
    return (final.reshape(B, C, H, W), weights.reshape(B, K, H, W))


def kernel(candidate_images, selection_scores, mask, partial_image):
    return _run(candidate_images, selection_scores, mask, partial_image)


# speedup vs baseline: 3.5364x; 2.7494x over previous
import functools

import jax
import jax.numpy as jnp
from jax import lax
from jax.experimental import pallas as pl
from jax.experimental.pallas import tpu as pltpu
from jax.experimental.pallas import tpu_sc as plsc

B, K, C, H, W = 8, 8, 3, 384, 384
NWORKERS = 32
ROWS_PER_WORKER = (B * H) // NWORKERS
TH, TW = 8, 128
NTROW = ROWS_PER_WORKER // TH
NTCOL = W // TW
NCHUNKS = NTROW * NTCOL
LANES = 16
SUBT = TW // LANES
WORKERS_PER_B = H // ROWS_PER_WORKER

f32 = jnp.float32


def _sc_body(cand_hbm, scores_hbm, mask_hbm, partial_hbm,
             final_hbm, weights_hbm,
             sv0, sv1, cv0, cv1, mv0, mv1, pv0, pv1,
             wv0, wv1, fv0, fv1,
             isem0, isem1, osem0, osem1):
    cid = lax.axis_index("c")
    sid = lax.axis_index("s")
    wid = cid * 16 + sid
    b = wid // WORKERS_PER_B
    h0 = (wid % WORKERS_PER_B) * ROWS_PER_WORKER

    svs, cvs, mvs, pvs = (sv0, sv1), (cv0, cv1), (mv0, mv1), (pv0, pv1)
    wvs, fvs = (wv0, wv1), (fv0, fv1)
    isems, osems = (isem0, isem1), (osem0, osem1)

    ones = jnp.full((LANES,), 1.0, f32)
    zeros = jnp.full((LANES,), 0.0, f32)

    def issue_in(t):
        slot = t % 2
        h = h0 + (t // NTCOL) * TH
        w = (t % NTCOL) * TW
        return [
            pltpu.async_copy(
                scores_hbm.at[b, :, pl.ds(h, TH), pl.ds(w, TW)],
                svs[slot], isems[slot]),
            pltpu.async_copy(
                cand_hbm.at[b, :, :, pl.ds(h, TH), pl.ds(w, TW)],
                cvs[slot], isems[slot]),
            pltpu.async_copy(
                mask_hbm.at[b, 0, pl.ds(h, TH), pl.ds(w, TW)],
                mvs[slot], isems[slot]),
            pltpu.async_copy(
                partial_hbm.at[b, :, pl.ds(h, TH), pl.ds(w, TW)],
                pvs[slot], isems[slot]),
        ]

    def issue_out(t):
        slot = t % 2
        h = h0 + (t // NTCOL) * TH
        w = (t % NTCOL) * TW
        return [
            pltpu.async_copy(
                wvs[slot], weights_hbm.at[b, :, pl.ds(h, TH), pl.ds(w, TW)],
                osems[slot]),
            pltpu.async_copy(
                fvs[slot], final_hbm.at[b, :, pl.ds(h, TH), pl.ds(w, TW)],
                osems[slot]),
        ]

    def compute(slot):
        sv, cv, mv, pv = svs[slot], cvs[slot], mvs[slot], pvs[slot]
        wv, fv = wvs[slot], fvs[slot]

        def row(r, carry0):
            def step(s, carry2):
                sl = pl.ds(s * LANES, LANES)
                m0 = sv[0, r, sl]
                idx = jnp.zeros((LANES,), jnp.int32)
                vals = [cv[0, c, r, sl] for c in range(C)]
                for k in range(1, K):
                    vk = sv[k, r, sl]
                    gt = vk > m0
                    m0 = jnp.where(gt, vk, m0)
                    idx = jnp.where(gt, jnp.full((LANES,), k, jnp.int32), idx)
                    for c in range(C):
                        vals[c] = jnp.where(gt, cv[k, c, r, sl], vals[c])
                for k in range(K):
                    kvec = jnp.full((LANES,), k, jnp.int32)
                    wv[k, r, sl] = jnp.where(idx == kvec, ones, zeros)
                mvec = mv[r, sl]
                fmv = 1.0 - mvec
                for c in range(C):
                    fv[c, r, sl] = pv[c, r, sl] * mvec + vals[c] * fmv
                return carry2

            lax.fori_loop(0, SUBT, step, 0)
            return carry0

        lax.fori_loop(0, TH, row, 0)

    out_descs = [None, None]
    in_descs = issue_in(0)
    for t in range(NCHUNKS):
        slot = t % 2
        next_descs = issue_in(t + 1) if t + 1 < NCHUNKS else None
        for d in in_descs:
            d.wait()
        if out_descs[slot] is not None:
            for d in out_descs[slot]:
                d.wait()
        compute(slot)
        out_descs[slot] = issue_out(t)
        in_descs = next_descs
    for slot in range(2):
        if out_descs[slot] is not None:
            for d in out_descs[slot]:
                d.wait()


@jax.jit
def _run(candidate_images, selection_scores, mask, partial_image):
    mesh = plsc.VectorSubcoreMesh(core_axis_name="c", subcore_axis_name="s")
    sc = functools.partial(
        pl.kernel,
        mesh=mesh,
        out_type=(
            jax.ShapeDtypeStruct((B, C, H, W), f32),
            jax.ShapeDtypeStruct((B, K, H, W), f32),
        ),
        compiler_params=pltpu.CompilerParams(use_tc_tiling_on_sc=True),
        scratch_types=[
            pltpu.VMEM((K, TH, TW), f32), pltpu.VMEM((K, TH, TW), f32),
            pltpu.VMEM((K, C, TH, TW), f32), pltpu.VMEM((K, C, TH, TW), f32),
            pltpu.VMEM((TH, TW), f32), pltpu.VMEM((TH, TW), f32),
            pltpu.VMEM((C, TH, TW), f32), pltpu.VMEM((C, TH, TW), f32),
            pltpu.VMEM((K, TH, TW), f32), pltpu.VMEM((K, TH, TW), f32),
            pltpu.VMEM((C, TH, TW), f32), pltpu.VMEM((C, TH, TW), f32),
            pltpu.SemaphoreType.DMA, pltpu.SemaphoreType.DMA,
            pltpu.SemaphoreType.DMA, pltpu.SemaphoreType.DMA,
        ],
    )(_sc_body)
    return sc(candidate_images, selection_scores, mask, partial_image)


def kernel(candidate_images, selection_scores, mask, partial_image):
    return _run(candidate_images, selection_scores, mask, partial_image)


# speedup vs baseline: 6.9340x; 1.9608x over previous
import functools

import jax
import jax.numpy as jnp
from jax.experimental import pallas as pl


def _body(cand_ref, scores_ref, mask_ref, partial_ref, final_ref, weights_ref):
    s = scores_ref[0]
    K = s.shape[0]
    best = jnp.argmax(s, axis=0)
    kidx = jax.lax.broadcasted_iota(jnp.int32, s.shape, 0)
    onehot = (kidx == best[None]).astype(jnp.float32)
    weights_ref[0] = onehot
    cand = cand_ref[0]
    sel = (cand * onehot[:, None]).sum(axis=0)
    m = mask_ref[0, 0]
    final_ref[0] = partial_ref[0] * m[None] + sel * (1.0 - m[None])


@functools.partial(jax.jit, static_argnames=("nh",))
def _run(candidate_images, selection_scores, mask, partial_image, nh=96):
    B, K, C, H, W = candidate_images.shape
    grid = (B, H // nh)
    out_shapes = (
        jax.ShapeDtypeStruct((B, C, H, W), jnp.float32),
        jax.ShapeDtypeStruct((B, K, H, W), jnp.float32),
    )
    return pl.pallas_call(
        _body,
        grid=grid,
        in_specs=[
            pl.BlockSpec((1, K, C, nh, W), lambda b, j: (b, 0, 0, j, 0)),
            pl.BlockSpec((1, K, nh, W), lambda b, j: (b, 0, j, 0)),
            pl.BlockSpec((1, 1, nh, W), lambda b, j: (b, 0, j, 0)),
            pl.BlockSpec((1, C, nh, W), lambda b, j: (b, 0, j, 0)),
        ],
        out_specs=(
            pl.BlockSpec((1, C, nh, W), lambda b, j: (b, 0, j, 0)),
            pl.BlockSpec((1, K, nh, W), lambda b, j: (b, 0, j, 0)),
        ),
        out_shape=out_shapes,
        compiler_params=__import__("jax.experimental.pallas.tpu", fromlist=["x"]).CompilerParams(dimension_semantics=("parallel", "arbitrary")),
    )(candidate_images, selection_scores, mask, partial_image)


def kernel(candidate_images, selection_scores, mask, partial_image):
    return _run(candidate_images, selection_scores, mask, partial_image)


# speedup vs baseline: 7.0751x; 1.0204x over previous
import functools

import jax
import jax.numpy as jnp
from jax.experimental import pallas as pl


def _body(cand_ref, scores_ref, mask_ref, partial_ref, final_ref, weights_ref):
    s = scores_ref[0]
    K = s.shape[0]
    best = jnp.argmax(s, axis=0)
    kidx = jax.lax.broadcasted_iota(jnp.int32, s.shape, 0)
    onehot = (kidx == best[None]).astype(jnp.float32)
    weights_ref[0] = onehot
    cand = cand_ref[0]
    sel = (cand * onehot[:, None]).sum(axis=0)
    m = mask_ref[0, 0]
    final_ref[0] = partial_ref[0] * m[None] + sel * (1.0 - m[None])


@functools.partial(jax.jit, static_argnames=("nh",))
def _run(candidate_images, selection_scores, mask, partial_image, nh=192):
    B, K, C, H, W = candidate_images.shape
    grid = (B, H // nh)
    out_shapes = (
        jax.ShapeDtypeStruct((B, C, H, W), jnp.float32),
        jax.ShapeDtypeStruct((B, K, H, W), jnp.float32),
    )
    return pl.pallas_call(
        _body,
        grid=grid,
        in_specs=[
            pl.BlockSpec((1, K, C, nh, W), lambda b, j: (b, 0, 0, j, 0)),
            pl.BlockSpec((1, K, nh, W), lambda b, j: (b, 0, j, 0)),
            pl.BlockSpec((1, 1, nh, W), lambda b, j: (b, 0, j, 0)),
            pl.BlockSpec((1, C, nh, W), lambda b, j: (b, 0, j, 0)),
        ],
        out_specs=(
            pl.BlockSpec((1, C, nh, W), lambda b, j: (b, 0, j, 0)),
            pl.BlockSpec((1, K, nh, W), lambda b, j: (b, 0, j, 0)),
        ),
        out_shape=out_shapes,
        compiler_params=__import__("jax.experimental.pallas.tpu", fromlist=["x"]).CompilerParams(dimension_semantics=("parallel", "arbitrary")),
    )(candidate_images, selection_scores, mask, partial_image)


def kernel(candidate_images, selection_scores, mask, partial_image):
    return _run(candidate_images, selection_scores, mask, partial_image)
